# baseline (speedup 1.0000x reference)
import functools

import jax
import jax.numpy as jnp
import numpy as np
from jax.experimental import pallas as pl
from jax.experimental.pallas import tpu as pltpu

B, H, S, D = 1, 12, 4096, 64
BLK = 64
NB = S // BLK
R = 3
NSEL = 3 + 2 + R
SCALE = 1.0 / np.sqrt(D)


def _attn_body(sel_ref, q_ref, k_ref, v_ref, o_ref):
    h = pl.program_id(0)
    n = pl.program_id(1)
    q = q_ref[0, 0]

    k_blocks = []
    v_blocks = []
    for j in range(NSEL):
        idx = sel_ref[h, n, j]
        off = idx * BLK
        k_blocks.append(k_ref[0, pl.ds(off, BLK), :])
        v_blocks.append(v_ref[0, pl.ds(off, BLK), :])
    ks = jnp.concatenate(k_blocks, axis=0)
    vs = jnp.concatenate(v_blocks, axis=0)

    scores = jax.lax.dot_general(
        q, ks, (((1,), (1,)), ((), ())),
        preferred_element_type=jnp.float32) * SCALE
    m = jnp.max(scores, axis=-1, keepdims=True)
    p = jnp.exp(scores - m)
    l = jnp.sum(p, axis=-1, keepdims=True)
    out = jax.lax.dot_general(
        p, vs, (((1,), (0,)), ((), ())),
        preferred_element_type=jnp.float32)
    o_ref[0, 0] = out / l


@jax.jit
def kernel(q, k, v, rand_attn):
    qh = q.reshape(H, NB, BLK, D)
    kh = k.reshape(H, S, D)
    vh = v.reshape(H, S, D)

    blk_ids = jnp.arange(NB, dtype=jnp.int32)
    win = jnp.stack([(blk_ids - 1) % NB, blk_ids, (blk_ids + 1) % NB], axis=-1)
    glob = jnp.broadcast_to(jnp.array([0, NB - 1], jnp.int32), (NB, 2))
    fixed = jnp.broadcast_to(
        jnp.concatenate([win, glob], axis=-1)[None], (H, NB, 5))
    sel = jnp.concatenate([fixed, rand_attn.astype(jnp.int32)], axis=-1)

    grid_spec = pltpu.PrefetchScalarGridSpec(
        num_scalar_prefetch=1,
        grid=(H, NB),
        in_specs=[
            pl.BlockSpec((1, 1, BLK, D), lambda h, n, sel: (h, n, 0, 0)),
            pl.BlockSpec((1, S, D), lambda h, n, sel: (h, 0, 0)),
            pl.BlockSpec((1, S, D), lambda h, n, sel: (h, 0, 0)),
        ],
        out_specs=pl.BlockSpec((1, 1, BLK, D), lambda h, n, sel: (h, n, 0, 0)),
    )
    out = pl.pallas_call(
        _attn_body,
        grid_spec=grid_spec,
        out_shape=jax.ShapeDtypeStruct((H, NB, BLK, D), jnp.float32),
    )(sel, qh, kh, vh)
    return out.reshape(B, H, S, D)


# speedup vs baseline: 1.8163x; 1.8163x over previous
import functools

import jax
import jax.numpy as jnp
import numpy as np
from jax.experimental import pallas as pl
from jax.experimental.pallas import tpu as pltpu

B, H, S, D = 1, 12, 4096, 64
BLK = 64
NB = S // BLK
R = 3
NSEL = 3 + 2 + R
SCALE = 1.0 / np.sqrt(D)


NQ = 4


def _attn_body(sel_ref, q_ref, k_ref, v_ref, o_ref):
    h = pl.program_id(0)
    g = pl.program_id(1)

    outs = []
    for i in range(NQ):
        n = g * NQ + i
        q = q_ref[0, pl.ds(i * BLK, BLK), :]

        k_blocks = []
        v_blocks = []
        for j in range(NSEL):
            idx = sel_ref[h, n, j]
            off = idx * BLK
            k_blocks.append(k_ref[0, pl.ds(off, BLK), :])
            v_blocks.append(v_ref[0, pl.ds(off, BLK), :])
        ks = jnp.concatenate(k_blocks, axis=0)
        vs = jnp.concatenate(v_blocks, axis=0)

        scores = jax.lax.dot_general(
            q, ks, (((1,), (1,)), ((), ())),
            preferred_element_type=jnp.float32) * SCALE
        m = jnp.max(scores, axis=-1, keepdims=True)
        p = jnp.exp(scores - m)
        l = jnp.sum(p, axis=-1, keepdims=True)
        out = jax.lax.dot_general(
            p.astype(jnp.bfloat16), vs.astype(jnp.bfloat16),
            (((1,), (0,)), ((), ())),
            preferred_element_type=jnp.float32)
        outs.append(out / l)
    o_ref[0] = jnp.concatenate(outs, axis=0)


@jax.jit
def kernel(q, k, v, rand_attn):
    qh = q.reshape(H, S, D)
    kh = k.reshape(H, S, D)
    vh = v.reshape(H, S, D)

    blk_ids = jnp.arange(NB, dtype=jnp.int32)
    win = jnp.stack([(blk_ids - 1) % NB, blk_ids, (blk_ids + 1) % NB], axis=-1)
    glob = jnp.broadcast_to(jnp.array([0, NB - 1], jnp.int32), (NB, 2))
    fixed = jnp.broadcast_to(
        jnp.concatenate([win, glob], axis=-1)[None], (H, NB, 5))
    sel = jnp.concatenate([fixed, rand_attn.astype(jnp.int32)], axis=-1)

    grid_spec = pltpu.PrefetchScalarGridSpec(
        num_scalar_prefetch=1,
        grid=(H, NB // NQ),
        in_specs=[
            pl.BlockSpec((1, NQ * BLK, D), lambda h, g, sel: (h, g, 0)),
            pl.BlockSpec((1, S, D), lambda h, g, sel: (h, 0, 0)),
            pl.BlockSpec((1, S, D), lambda h, g, sel: (h, 0, 0)),
        ],
        out_specs=pl.BlockSpec((1, NQ * BLK, D), lambda h, g, sel: (h, g, 0)),
    )
    out = pl.pallas_call(
        _attn_body,
        grid_spec=grid_spec,
        out_shape=jax.ShapeDtypeStruct((H, S, D), jnp.float32),
    )(sel, qh, kh, vh)
    return out.reshape(B, H, S, D)


# speedup vs baseline: 1.9839x; 1.0923x over previous
import functools

import jax
import jax.numpy as jnp
import numpy as np
from jax.experimental import pallas as pl
from jax.experimental.pallas import tpu as pltpu

B, H, S, D = 1, 12, 4096, 64
BLK = 64
NB = S // BLK
R = 3
NSEL = 3 + 2 + R
SCALE = 1.0 / np.sqrt(D)


NQ = 8


def _attn_body(sel_ref, q_ref, k_ref, v_ref, o_ref):
    h = pl.program_id(0)
    g = pl.program_id(1)

    outs = []
    for i in range(NQ):
        n = g * NQ + i
        q = q_ref[0, pl.ds(i * BLK, BLK), :]

        k_blocks = []
        v_blocks = []
        for j in range(NSEL):
            idx = sel_ref[h, n, j]
            off = idx * BLK
            k_blocks.append(k_ref[0, pl.ds(off, BLK), :])
            v_blocks.append(v_ref[0, pl.ds(off, BLK), :])
        ks = jnp.concatenate(k_blocks, axis=0)
        vs = jnp.concatenate(v_blocks, axis=0)

        scores = jax.lax.dot_general(
            q, ks, (((1,), (1,)), ((), ())),
            preferred_element_type=jnp.float32) * SCALE
        m = jnp.max(scores, axis=-1, keepdims=True)
        p = jnp.exp(scores - m)
        l = jnp.sum(p, axis=-1, keepdims=True)
        out = jax.lax.dot_general(
            p.astype(jnp.bfloat16), vs.astype(jnp.bfloat16),
            (((1,), (0,)), ((), ())),
            preferred_element_type=jnp.float32)
        outs.append(out / l)
    o_ref[0] = jnp.concatenate(outs, axis=0)


@jax.jit
def kernel(q, k, v, rand_attn):
    qh = q.reshape(H, S, D)
    kh = k.reshape(H, S, D)
    vh = v.reshape(H, S, D)

    blk_ids = jnp.arange(NB, dtype=jnp.int32)
    win = jnp.stack([(blk_ids - 1) % NB, blk_ids, (blk_ids + 1) % NB], axis=-1)
    glob = jnp.broadcast_to(jnp.array([0, NB - 1], jnp.int32), (NB, 2))
    fixed = jnp.broadcast_to(
        jnp.concatenate([win, glob], axis=-1)[None], (H, NB, 5))
    sel = jnp.concatenate([fixed, rand_attn.astype(jnp.int32)], axis=-1)

    grid_spec = pltpu.PrefetchScalarGridSpec(
        num_scalar_prefetch=1,
        grid=(H, NB // NQ),
        in_specs=[
            pl.BlockSpec((1, NQ * BLK, D), lambda h, g, sel: (h, g, 0)),
            pl.BlockSpec((1, S, D), lambda h, g, sel: (h, 0, 0)),
            pl.BlockSpec((1, S, D), lambda h, g, sel: (h, 0, 0)),
        ],
        out_specs=pl.BlockSpec((1, NQ * BLK, D), lambda h, g, sel: (h, g, 0)),
    )
    out = pl.pallas_call(
        _attn_body,
        grid_spec=grid_spec,
        out_shape=jax.ShapeDtypeStruct((H, S, D), jnp.float32),
    )(sel, qh, kh, vh)
    return out.reshape(B, H, S, D)


# speedup vs baseline: 1.9850x; 1.0006x over previous
import functools

import jax
import jax.numpy as jnp
import numpy as np
from jax.experimental import pallas as pl
from jax.experimental.pallas import tpu as pltpu

B, H, S, D = 1, 12, 4096, 64
BLK = 64
NB = S // BLK
R = 3
NSEL = 3 + 2 + R
SCALE = 1.0 / np.sqrt(D)


NQ = 8


def _attn_body(sel_ref, q_ref, k_ref, v_ref, o_ref):
    h = pl.program_id(0)
    g = pl.program_id(1)

    outs = []
    for i in range(NQ):
        n = g * NQ + i
        q = q_ref[0, pl.ds(i * BLK, BLK), :]

        k_blocks = []
        v_blocks = []
        for j in range(NSEL):
            idx = sel_ref[h, n, j]
            off = idx * BLK
            k_blocks.append(k_ref[0, pl.ds(off, BLK), :])
            v_blocks.append(v_ref[0, pl.ds(off, BLK), :])
        ks = jnp.concatenate(k_blocks, axis=0)
        vs = jnp.concatenate(v_blocks, axis=0)

        scores = jax.lax.dot_general(
            q.astype(jnp.bfloat16), ks.astype(jnp.bfloat16),
            (((1,), (1,)), ((), ())),
            preferred_element_type=jnp.float32) * SCALE
        m = jnp.max(scores, axis=-1, keepdims=True)
        p = jnp.exp(scores - m)
        l = jnp.sum(p, axis=-1, keepdims=True)
        out = jax.lax.dot_general(
            p.astype(jnp.bfloat16), vs.astype(jnp.bfloat16),
            (((1,), (0,)), ((), ())),
            preferred_element_type=jnp.float32)
        outs.append(out / l)
    o_ref[0] = jnp.concatenate(outs, axis=0)


@jax.jit
def kernel(q, k, v, rand_attn):
    qh = q.reshape(H, S, D)
    kh = k.reshape(H, S, D)
    vh = v.reshape(H, S, D)

    blk_ids = jnp.arange(NB, dtype=jnp.int32)
    win = jnp.stack([(blk_ids - 1) % NB, blk_ids, (blk_ids + 1) % NB], axis=-1)
    glob = jnp.broadcast_to(jnp.array([0, NB - 1], jnp.int32), (NB, 2))
    fixed = jnp.broadcast_to(
        jnp.concatenate([win, glob], axis=-1)[None], (H, NB, 5))
    sel = jnp.concatenate([fixed, rand_attn.astype(jnp.int32)], axis=-1)

    grid_spec = pltpu.PrefetchScalarGridSpec(
        num_scalar_prefetch=1,
        grid=(H, NB // NQ),
        in_specs=[
            pl.BlockSpec((1, NQ * BLK, D), lambda h, g, sel: (h, g, 0)),
            pl.BlockSpec((1, S, D), lambda h, g, sel: (h, 0, 0)),
            pl.BlockSpec((1, S, D), lambda h, g, sel: (h, 0, 0)),
        ],
        out_specs=pl.BlockSpec((1, NQ * BLK, D), lambda h, g, sel: (h, g, 0)),
    )
    out = pl.pallas_call(
        _attn_body,
        grid_spec=grid_spec,
        out_shape=jax.ShapeDtypeStruct((H, S, D), jnp.float32),
    )(sel, qh, kh, vh)
    return out.reshape(B, H, S, D)
